# baseline (speedup 1.0000x reference)
import functools

import jax
import jax.numpy as jnp
from jax import lax
from jax.experimental import pallas as pl
from jax.experimental.pallas import tpu as pltpu
from jax.experimental.pallas import tpu_sc as plsc

D_MODEL = 64
SCALE = 8.0
NUM_WORKERS = 32
CHUNK = 128
LANES = 16
NBUF = 2


def _gather_scale_t(idx, lut, batch, hist):
    b_per_w = batch // NUM_WORKERS
    n_idx = b_per_w * hist
    bchunks = b_per_w // CHUNK
    steps = hist * bchunks
    outer = steps // NBUF

    mesh = plsc.VectorSubcoreMesh(core_axis_name="c", subcore_axis_name="s")

    @functools.partial(
        pl.kernel,
        mesh=mesh,
        out_type=jax.ShapeDtypeStruct((hist, D_MODEL, batch), jnp.float32),
        scratch_types=[
            pltpu.VMEM((n_idx,), jnp.int32),
            pltpu.VMEM((hist, b_per_w), jnp.int32),
            pltpu.VMEM((NBUF, CHUNK, D_MODEL), jnp.float32),
            pltpu.VMEM((NBUF, D_MODEL, CHUNK), jnp.float32),
            [pltpu.SemaphoreType.DMA] * NBUF,
            [pltpu.SemaphoreType.DMA] * NBUF,
        ],
        compiler_params=pltpu.CompilerParams(
            use_tc_tiling_on_sc=False, needs_layout_passes=False
        ),
    )
    def k(lut_hbm, idx_hbm, out_hbm, idx_raw, idx_t, gbufs, tbufs, gsems, wsems):
        wid = lax.axis_index("s") * 2 + lax.axis_index("c")
        b0 = wid * b_per_w
        pltpu.sync_copy(idx_hbm.at[pl.ds(b0 * hist, n_idx)], idx_raw)

        iota = lax.iota(jnp.int32, LANES)

        def regroup_h(h, carry):
            def regroup_j(j, carry2):
                flat = (iota + j * LANES) * hist + h
                v = plsc.load_gather(idx_raw, [flat])
                idx_t[h, pl.ds(j * LANES, LANES)] = v
                return carry2

            lax.fori_loop(0, b_per_w // LANES, regroup_j, 0)
            return carry

        lax.fori_loop(0, hist, regroup_h, 0)

        def start_gather(s, b):
            h = s // bchunks
            bb = s % bchunks
            pltpu.async_copy(
                lut_hbm.at[idx_t.at[h, pl.ds(bb * CHUNK, CHUNK)]],
                gbufs.at[b],
                gsems[b],
            )

        for b in range(NBUF):
            start_gather(b, b)

        def outer_body(s2, carry):
            for b in range(NBUF):
                s = s2 * NBUF + b
                h = s // bchunks
                bb = s % bchunks
                gb = gbufs.at[b]
                tb = tbufs.at[b]
                pltpu.make_async_copy(
                    lut_hbm.at[idx_t.at[0, pl.ds(0, CHUNK)]], gb, gsems[b]
                ).wait()

                @pl.when(s2 > 0)
                def _():
                    pltpu.make_async_copy(
                        tb, out_hbm.at[0, :, pl.ds(0, CHUNK)], wsems[b]
                    ).wait()

                def tr_d(d, carry2):
                    cols = jnp.full((LANES,), d, jnp.int32)
                    for j in range(CHUNK // LANES):
                        rows = iota + j * LANES
                        v = plsc.load_gather(gb, [rows, cols]) * SCALE
                        tb[d, pl.ds(j * LANES, LANES)] = v
                    return carry2

                lax.fori_loop(0, D_MODEL, tr_d, 0)

                @pl.when(s2 < outer - 1)
                def _():
                    start_gather(s + NBUF, b)

                pltpu.async_copy(
                    tb,
                    out_hbm.at[h, :, pl.ds(b0 + bb * CHUNK, CHUNK)],
                    wsems[b],
                )
            return carry

        lax.fori_loop(0, outer, outer_body, 0)

        for b in range(NBUF):
            pltpu.make_async_copy(
                tbufs.at[b], out_hbm.at[0, :, pl.ds(0, CHUNK)], wsems[b]
            ).wait()

    return k(lut, idx)


def kernel(x, lut):
    batch, hist = x.shape
    idx = x.reshape(batch * hist)
    out_t = _gather_scale_t(idx, lut, batch, hist)
    return jnp.transpose(out_t, (2, 0, 1))


# speedup vs baseline: 1.9092x; 1.9092x over previous
import functools

import jax
import jax.numpy as jnp
from jax import lax
from jax.experimental import pallas as pl
from jax.experimental.pallas import tpu as pltpu
from jax.experimental.pallas import tpu_sc as plsc

D_MODEL = 64
SCALE = 8.0
NUM_WORKERS = 16
CHUNK = 128
LANES = 16
NBUF = 2
RAW = 25600


def _gather_scale_t(idx, lut, batch, hist):
    b_per_w = batch // NUM_WORKERS
    bchunks = b_per_w // CHUNK
    steps = hist * bchunks
    outer = steps // NBUF
    halves = (b_per_w * hist) // RAW
    b_half = b_per_w // halves

    mesh = plsc.VectorSubcoreMesh(
        core_axis_name="c", subcore_axis_name="s", num_cores=1
    )

    @functools.partial(
        pl.kernel,
        mesh=mesh,
        out_type=jax.ShapeDtypeStruct((hist, D_MODEL, batch), jnp.float32),
        scratch_types=[
            pltpu.VMEM((RAW,), jnp.int32),
            pltpu.VMEM((hist, b_per_w), jnp.int32),
            pltpu.VMEM((NBUF, CHUNK, D_MODEL), jnp.float32),
            pltpu.VMEM((NBUF, D_MODEL, CHUNK + 1), jnp.float32),
            [pltpu.SemaphoreType.DMA] * NBUF,
            [pltpu.SemaphoreType.DMA] * NBUF,
        ],
        compiler_params=pltpu.CompilerParams(
            use_tc_tiling_on_sc=False, needs_layout_passes=False
        ),
    )
    def k(lut_hbm, idx_hbm, out_hbm, idx_raw, idx_t, gbufs, tbufs, gsems, wsems):
        wid = lax.axis_index("s")
        b0 = wid * b_per_w

        iota = lax.iota(jnp.int32, LANES)

        for q in range(halves):
            pltpu.sync_copy(
                idx_hbm.at[pl.ds((b0 + q * b_half) * hist, RAW)], idx_raw
            )

            def regroup_h(h, carry):
                def regroup_j(j, carry2):
                    flat = (iota + j * LANES) * hist + h
                    v = plsc.load_gather(idx_raw, [flat])
                    idx_t[h, pl.ds(q * b_half + j * LANES, LANES)] = v
                    return carry2

                lax.fori_loop(0, b_half // LANES, regroup_j, 0)
                return carry

            lax.fori_loop(0, hist, regroup_h, 0)

        def start_gather(s, b):
            h = s // bchunks
            bb = s % bchunks
            pltpu.async_copy(
                lut_hbm.at[idx_t.at[h, pl.ds(bb * CHUNK, CHUNK)]],
                gbufs.at[b],
                gsems[b],
            )

        for b in range(NBUF):
            start_gather(b, b)

        def outer_body(s2, carry):
            for b in range(NBUF):
                s = s2 * NBUF + b
                h = s // bchunks
                bb = s % bchunks
                gb = gbufs.at[b]
                tb = tbufs.at[b]
                pltpu.make_async_copy(
                    lut_hbm.at[idx_t.at[0, pl.ds(0, CHUNK)]], gb, gsems[b]
                ).wait()

                @pl.when(s2 > 0)
                def _():
                    pltpu.make_async_copy(
                        tbufs.at[b, :, pl.ds(0, CHUNK)],
                        out_hbm.at[0, :, pl.ds(0, CHUNK)],
                        wsems[b],
                    ).wait()

                @plsc.parallel_loop(0, CHUNK, step=1, unroll=4)
                def tr_r(r):
                    cols = jnp.full((LANES,), r, jnp.int32)
                    for j in range(D_MODEL // LANES):
                        rows = iota + j * LANES
                        v = gb[r, pl.ds(j * LANES, LANES)] * SCALE
                        plsc.store_scatter(tb, [rows, cols], v)

                @pl.when(s2 < outer - 1)
                def _():
                    start_gather(s + NBUF, b)

                pltpu.async_copy(
                    tbufs.at[b, :, pl.ds(0, CHUNK)],
                    out_hbm.at[h, :, pl.ds(b0 + bb * CHUNK, CHUNK)],
                    wsems[b],
                )
            return carry

        lax.fori_loop(0, outer, outer_body, 0)

        for b in range(NBUF):
            pltpu.make_async_copy(
                tbufs.at[b, :, pl.ds(0, CHUNK)],
                out_hbm.at[0, :, pl.ds(0, CHUNK)],
                wsems[b],
            ).wait()

    return k(lut, idx)


def kernel(x, lut):
    batch, hist = x.shape
    idx = x.reshape(batch * hist)
    out_t = _gather_scale_t(idx, lut, batch, hist)
    return jnp.transpose(out_t, (2, 0, 1))


# speedup vs baseline: 2.2120x; 1.1586x over previous
import functools

import jax
import jax.numpy as jnp
from jax import lax
from jax.experimental import pallas as pl
from jax.experimental.pallas import tpu as pltpu
from jax.experimental.pallas import tpu_sc as plsc

D_MODEL = 64
SCALE = 8.0
NUM_WORKERS = 32
CHUNK = 256
LANES = 16
NBUF = 2


def _gather_scale_t(idx, lut, batch, hist):
    b_per_w = batch // NUM_WORKERS
    n_idx = b_per_w * hist
    bchunks = b_per_w // CHUNK
    steps = hist * bchunks
    outer = steps // NBUF

    mesh = plsc.VectorSubcoreMesh(core_axis_name="c", subcore_axis_name="s")

    @functools.partial(
        pl.kernel,
        mesh=mesh,
        out_type=jax.ShapeDtypeStruct((hist, D_MODEL, batch), jnp.float32),
        scratch_types=[
            pltpu.VMEM((n_idx,), jnp.int32),
            pltpu.VMEM((hist, b_per_w), jnp.int32),
            pltpu.VMEM((NBUF, CHUNK, D_MODEL), jnp.float32),
            pltpu.VMEM((NBUF, D_MODEL, CHUNK + 1), jnp.float32),
            [pltpu.SemaphoreType.DMA] * NBUF,
            [pltpu.SemaphoreType.DMA] * NBUF,
        ],
        compiler_params=pltpu.CompilerParams(
            use_tc_tiling_on_sc=False, needs_layout_passes=False
        ),
    )
    def k(lut_hbm, idx_hbm, out_hbm, idx_raw, idx_t, gbufs, tbufs, gsems, wsems):
        wid = lax.axis_index("s") * 2 + lax.axis_index("c")
        b0 = wid * b_per_w
        pltpu.sync_copy(idx_hbm.at[pl.ds(b0 * hist, n_idx)], idx_raw)

        iota = lax.iota(jnp.int32, LANES)

        def regroup_h(h, carry):
            def regroup_j(j, carry2):
                flat = (iota + j * LANES) * hist + h
                v = plsc.load_gather(idx_raw, [flat])
                idx_t[h, pl.ds(j * LANES, LANES)] = v
                return carry2

            lax.fori_loop(0, b_per_w // LANES, regroup_j, 0)
            return carry

        lax.fori_loop(0, hist, regroup_h, 0)

        def start_gather(s, b):
            h = s // bchunks
            bb = s % bchunks
            pltpu.async_copy(
                lut_hbm.at[idx_t.at[h, pl.ds(bb * CHUNK, CHUNK)]],
                gbufs.at[b],
                gsems[b],
            )

        for b in range(NBUF):
            start_gather(b, b)

        def outer_body(s2, carry):
            for b in range(NBUF):
                s = s2 * NBUF + b
                h = s // bchunks
                bb = s % bchunks
                gb = gbufs.at[b]
                tb = tbufs.at[b]
                pltpu.make_async_copy(
                    lut_hbm.at[idx_t.at[0, pl.ds(0, CHUNK)]], gb, gsems[b]
                ).wait()

                @pl.when(s2 > 0)
                def _():
                    pltpu.make_async_copy(
                        tbufs.at[b, :, pl.ds(0, CHUNK)],
                        out_hbm.at[0, :, pl.ds(0, CHUNK)],
                        wsems[b],
                    ).wait()

                @plsc.parallel_loop(0, CHUNK, step=1, unroll=4)
                def tr_r(r):
                    cols = jnp.full((LANES,), r, jnp.int32)
                    for j in range(D_MODEL // LANES):
                        rows = iota + j * LANES
                        v = gb[r, pl.ds(j * LANES, LANES)] * SCALE
                        plsc.store_scatter(tb, [rows, cols], v)

                @pl.when(s2 < outer - 1)
                def _():
                    start_gather(s + NBUF, b)

                pltpu.async_copy(
                    tbufs.at[b, :, pl.ds(0, CHUNK)],
                    out_hbm.at[h, :, pl.ds(b0 + bb * CHUNK, CHUNK)],
                    wsems[b],
                )
            return carry

        lax.fori_loop(0, outer, outer_body, 0)

        for b in range(NBUF):
            pltpu.make_async_copy(
                tbufs.at[b, :, pl.ds(0, CHUNK)],
                out_hbm.at[0, :, pl.ds(0, CHUNK)],
                wsems[b],
            ).wait()

    return k(lut, idx)


def kernel(x, lut):
    batch, hist = x.shape
    idx = x.reshape(batch * hist)
    out_t = _gather_scale_t(idx, lut, batch, hist)
    return jnp.transpose(out_t, (2, 0, 1))


# speedup vs baseline: 2.2134x; 1.0006x over previous
import functools

import jax
import jax.numpy as jnp
from jax import lax
from jax.experimental import pallas as pl
from jax.experimental.pallas import tpu as pltpu
from jax.experimental.pallas import tpu_sc as plsc

D_MODEL = 64
SCALE = 8.0
NUM_WORKERS = 32
CHUNK = 128
LANES = 16
NBUF = 4


def _gather_scale_t(idx, lut, batch, hist):
    b_per_w = batch // NUM_WORKERS
    n_idx = b_per_w * hist
    bchunks = b_per_w // CHUNK
    steps = hist * bchunks
    outer = steps // NBUF

    mesh = plsc.VectorSubcoreMesh(core_axis_name="c", subcore_axis_name="s")

    @functools.partial(
        pl.kernel,
        mesh=mesh,
        out_type=jax.ShapeDtypeStruct((hist, D_MODEL, batch), jnp.float32),
        scratch_types=[
            pltpu.VMEM((n_idx,), jnp.int32),
            pltpu.VMEM((hist, b_per_w), jnp.int32),
            pltpu.VMEM((NBUF, CHUNK, D_MODEL), jnp.float32),
            pltpu.VMEM((NBUF, D_MODEL, CHUNK + 1), jnp.float32),
            [pltpu.SemaphoreType.DMA] * NBUF,
            [pltpu.SemaphoreType.DMA] * NBUF,
        ],
        compiler_params=pltpu.CompilerParams(
            use_tc_tiling_on_sc=False, needs_layout_passes=False
        ),
    )
    def k(lut_hbm, idx_hbm, out_hbm, idx_raw, idx_t, gbufs, tbufs, gsems, wsems):
        wid = lax.axis_index("s") * 2 + lax.axis_index("c")
        b0 = wid * b_per_w
        pltpu.sync_copy(idx_hbm.at[pl.ds(b0 * hist, n_idx)], idx_raw)

        iota = lax.iota(jnp.int32, LANES)

        def regroup_h(h, carry):
            def regroup_j(j, carry2):
                flat = (iota + j * LANES) * hist + h
                v = plsc.load_gather(idx_raw, [flat])
                idx_t[h, pl.ds(j * LANES, LANES)] = v
                return carry2

            lax.fori_loop(0, b_per_w // LANES, regroup_j, 0)
            return carry

        lax.fori_loop(0, hist, regroup_h, 0)

        def start_gather(s, b):
            h = s // bchunks
            bb = s % bchunks
            pltpu.async_copy(
                lut_hbm.at[idx_t.at[h, pl.ds(bb * CHUNK, CHUNK)]],
                gbufs.at[b],
                gsems[b],
            )

        for b in range(NBUF):
            start_gather(b, b)

        def outer_body(s2, carry):
            for b in range(NBUF):
                s = s2 * NBUF + b
                h = s // bchunks
                bb = s % bchunks
                gb = gbufs.at[b]
                tb = tbufs.at[b]
                pltpu.make_async_copy(
                    lut_hbm.at[idx_t.at[0, pl.ds(0, CHUNK)]], gb, gsems[b]
                ).wait()

                @pl.when(s2 > 0)
                def _():
                    pltpu.make_async_copy(
                        tbufs.at[b, :, pl.ds(0, CHUNK)],
                        out_hbm.at[0, :, pl.ds(0, CHUNK)],
                        wsems[b],
                    ).wait()

                @plsc.parallel_loop(0, CHUNK, step=1, unroll=4)
                def tr_r(r):
                    cols = jnp.full((LANES,), r, jnp.int32)
                    for j in range(D_MODEL // LANES):
                        rows = iota + j * LANES
                        v = gb[r, pl.ds(j * LANES, LANES)] * SCALE
                        plsc.store_scatter(tb, [rows, cols], v)

                @pl.when(s2 < outer - 1)
                def _():
                    start_gather(s + NBUF, b)

                pltpu.async_copy(
                    tbufs.at[b, :, pl.ds(0, CHUNK)],
                    out_hbm.at[h, :, pl.ds(b0 + bb * CHUNK, CHUNK)],
                    wsems[b],
                )
            return carry

        lax.fori_loop(0, outer, outer_body, 0)

        for b in range(NBUF):
            pltpu.make_async_copy(
                tbufs.at[b, :, pl.ds(0, CHUNK)],
                out_hbm.at[0, :, pl.ds(0, CHUNK)],
                wsems[b],
            ).wait()

    return k(lut, idx)


def kernel(x, lut):
    batch, hist = x.shape
    idx = x.reshape(batch * hist)
    out_t = _gather_scale_t(idx, lut, batch, hist)
    return jnp.transpose(out_t, (2, 0, 1))


# speedup vs baseline: 2.2427x; 1.0132x over previous
import functools

import jax
import jax.numpy as jnp
from jax import lax
from jax.experimental import pallas as pl
from jax.experimental.pallas import tpu as pltpu
from jax.experimental.pallas import tpu_sc as plsc

D_MODEL = 64
SCALE = 8.0
NUM_WORKERS = 32
CHUNK = 128
LANES = 16
HBUF = 2


def _gather_scale_t(idx_t_arr, lut, batch, hist):
    b_per_w = batch // NUM_WORKERS
    bchunks = b_per_w // CHUNK

    mesh = plsc.VectorSubcoreMesh(core_axis_name="c", subcore_axis_name="s")

    @functools.partial(
        pl.kernel,
        mesh=mesh,
        out_type=jax.ShapeDtypeStruct((hist, D_MODEL, batch), jnp.float32),
        scratch_types=[
            pltpu.VMEM((hist, b_per_w), jnp.int32),
            pltpu.VMEM((bchunks, CHUNK, D_MODEL), jnp.float32),
            pltpu.VMEM((HBUF, D_MODEL, b_per_w + 1), jnp.float32),
            [pltpu.SemaphoreType.DMA] * bchunks,
            [pltpu.SemaphoreType.DMA] * HBUF,
        ],
        compiler_params=pltpu.CompilerParams(
            use_tc_tiling_on_sc=False, needs_layout_passes=False
        ),
    )
    def k(lut_hbm, idx_hbm, out_hbm, idx_t, gbufs, hbufs, gsems, wsems):
        wid = lax.axis_index("s") * 2 + lax.axis_index("c")
        b0 = wid * b_per_w
        pltpu.sync_copy(idx_hbm.at[:, pl.ds(b0, b_per_w)], idx_t)

        iota = lax.iota(jnp.int32, LANES)

        def start_gather(h, bb):
            pltpu.async_copy(
                lut_hbm.at[idx_t.at[h, pl.ds(bb * CHUNK, CHUNK)]],
                gbufs.at[bb],
                gsems[bb],
            )

        for bb in range(bchunks):
            start_gather(0, bb)

        def outer_body(h2, carry):
            for hh in range(HBUF):
                h = h2 * HBUF + hh
                hb = hbufs.at[hh]

                @pl.when(h2 > 0)
                def _():
                    pltpu.make_async_copy(
                        hbufs.at[hh, :, pl.ds(0, b_per_w)],
                        out_hbm.at[0, :, pl.ds(0, b_per_w)],
                        wsems[hh],
                    ).wait()

                for bb in range(bchunks):
                    gb = gbufs.at[bb]
                    pltpu.make_async_copy(
                        lut_hbm.at[idx_t.at[0, pl.ds(0, CHUNK)]],
                        gb,
                        gsems[bb],
                    ).wait()

                    @plsc.parallel_loop(0, CHUNK, step=1, unroll=4)
                    def tr_r(r):
                        cols = jnp.full((LANES,), bb * CHUNK + r, jnp.int32)
                        for j in range(D_MODEL // LANES):
                            rows = iota + j * LANES
                            v = gb[r, pl.ds(j * LANES, LANES)] * SCALE
                            plsc.store_scatter(hb, [rows, cols], v)

                    @pl.when(h < hist - 1)
                    def _():
                        start_gather(h + 1, bb)

                pltpu.async_copy(
                    hbufs.at[hh, :, pl.ds(0, b_per_w)],
                    out_hbm.at[h, :, pl.ds(b0, b_per_w)],
                    wsems[hh],
                )
            return carry

        lax.fori_loop(0, hist // HBUF, outer_body, 0)

        for hh in range(HBUF):
            pltpu.make_async_copy(
                hbufs.at[hh, :, pl.ds(0, b_per_w)],
                out_hbm.at[0, :, pl.ds(0, b_per_w)],
                wsems[hh],
            ).wait()

    return k(lut, idx_t_arr)


def kernel(x, lut):
    batch, hist = x.shape
    xt = jnp.transpose(x)
    out_t = _gather_scale_t(xt, lut, batch, hist)
    return jnp.transpose(out_t, (2, 0, 1))


# speedup vs baseline: 2.8851x; 1.2865x over previous
import functools

import jax
import jax.numpy as jnp
from jax import lax
from jax.experimental import pallas as pl
from jax.experimental.pallas import tpu as pltpu
from jax.experimental.pallas import tpu_sc as plsc

D_MODEL = 64
SCALE = 8.0
NUM_WORKERS = 32
CHUNK = 128
LANES = 16
HBUF = 2


def _gather_scale_t(idx_t_arr, lut, batch, hist):
    b_per_w = batch // NUM_WORKERS
    bchunks = b_per_w // CHUNK

    mesh = plsc.VectorSubcoreMesh(core_axis_name="c", subcore_axis_name="s")

    @functools.partial(
        pl.kernel,
        mesh=mesh,
        out_type=jax.ShapeDtypeStruct(
            (hist, D_MODEL // 8, batch // CHUNK, 8, CHUNK), jnp.float32
        ),
        scratch_types=[
            pltpu.VMEM((hist, b_per_w), jnp.int32),
            pltpu.VMEM((bchunks, CHUNK, D_MODEL), jnp.float32),
            pltpu.VMEM((HBUF, D_MODEL // 8, 8, b_per_w + 1), jnp.float32),
            [pltpu.SemaphoreType.DMA] * bchunks,
            [pltpu.SemaphoreType.DMA] * HBUF,
        ],
        compiler_params=pltpu.CompilerParams(
            use_tc_tiling_on_sc=False, needs_layout_passes=False
        ),
    )
    def k(lut_hbm, idx_hbm, out_hbm, idx_t, gbufs, hbufs, gsems, wsems):
        wid = lax.axis_index("s") * 2 + lax.axis_index("c")
        b0 = wid * b_per_w
        bt0 = wid * bchunks
        pltpu.sync_copy(idx_hbm.at[:, pl.ds(b0, b_per_w)], idx_t)

        iota = lax.iota(jnp.int32, LANES)

        def start_gather(h, bb):
            pltpu.async_copy(
                lut_hbm.at[idx_t.at[h, pl.ds(bb * CHUNK, CHUNK)]],
                gbufs.at[bb],
                gsems[bb],
            )

        for bb in range(bchunks):
            start_gather(0, bb)

        def outer_body(h2, carry):
            for hh in range(HBUF):
                h = h2 * HBUF + hh
                hb = hbufs.at[hh]

                @pl.when(h2 > 0)
                def _():
                    for i in range(bchunks):
                        pltpu.make_async_copy(
                            hbufs.at[hh, :, :, pl.ds(0, CHUNK)],
                            out_hbm.at[0, :, 0, :, :],
                            wsems[hh],
                        ).wait()

                for bb in range(bchunks):
                    gb = gbufs.at[bb]
                    pltpu.make_async_copy(
                        lut_hbm.at[idx_t.at[0, pl.ds(0, CHUNK)]],
                        gb,
                        gsems[bb],
                    ).wait()

                    @plsc.parallel_loop(0, CHUNK, step=1, unroll=4)
                    def tr_r(r):
                        cols = jnp.full((LANES,), bb * CHUNK + r, jnp.int32)
                        for j in range(D_MODEL // LANES):
                            rows = iota + j * LANES
                            v = gb[r, pl.ds(j * LANES, LANES)] * SCALE
                            plsc.store_scatter(
                                hb, [rows >> 3, rows & 7, cols], v
                            )

                    @pl.when(h < hist - 1)
                    def _():
                        start_gather(h + 1, bb)

                for i in range(bchunks):
                    pltpu.async_copy(
                        hbufs.at[hh, :, :, pl.ds(i * CHUNK, CHUNK)],
                        out_hbm.at[h, :, bt0 + i, :, :],
                        wsems[hh],
                    )
            return carry

        lax.fori_loop(0, hist // HBUF, outer_body, 0)

        for hh in range(HBUF):
            for i in range(bchunks):
                pltpu.make_async_copy(
                    hbufs.at[hh, :, :, pl.ds(0, CHUNK)],
                    out_hbm.at[0, :, 0, :, :],
                    wsems[hh],
                ).wait()

    return k(lut, idx_t_arr)


def kernel(x, lut):
    batch, hist = x.shape
    xt = jnp.transpose(x)
    out5 = _gather_scale_t(xt, lut, batch, hist)
    return jnp.transpose(out5, (2, 4, 0, 1, 3)).reshape(batch, hist, D_MODEL)
